# baseline (speedup 1.0000x reference)
import functools

import jax
import jax.numpy as jnp
from jax import lax
from jax.experimental import pallas as pl
from jax.experimental.pallas import tpu as pltpu
from jax.experimental.pallas import tpu_sc as plsc

N = 10000
D_IN = 256
D_HID = 256
D_OUT = 128
E = 160000

_CHUNK = 80
_NSUB = 16
_NREAL = E // _CHUNK
_DEPTH = 4
_IDX_SLOTS = 2 * _DEPTH
_G_AGG = 128
_NBODY = _G_AGG // _IDX_SLOTS
_DEG_DEPTH = 3
_NMACRO = 42
_G_IDX = _G_AGG + _DEPTH
_PAD = _NSUB * _G_IDX * _CHUNK - E
_ACC_PAD_ROWS = 8
_ROWS_A = 632
_ROWS_LAST = N - 15 * _ROWS_A

_ROW_BLOCK = 1000


def _pad_edges(adj):
    i = jnp.arange(_PAD, dtype=jnp.int32)
    return jnp.concatenate([adj.reshape(2 * E), N + (i % _ACC_PAD_ROWS)])


def _src_slice(h, s, g):
    return h.at[pl.ds((g * _NSUB + s) * _CHUNK, _CHUNK)]


def _dst_slice(h, s, g):
    return h.at[pl.ds(E + (g * _NSUB + s) * _CHUNK, _CHUNK)]


def _mesh():
    return plsc.VectorSubcoreMesh(core_axis_name="c", subcore_axis_name="s")


def _node_range_copy(s, copy_fn):
    @pl.when(s < 15)
    def _():
        copy_fn(s * _ROWS_A, _ROWS_A)

    @pl.when(s == 15)
    def _():
        copy_fn(15 * _ROWS_A, _ROWS_LAST)


def _sc_degrees(eidx1, eidx2):
    @functools.partial(
        pl.kernel,
        out_type=[jax.ShapeDtypeStruct((N,), jnp.float32)] * 2,
        mesh=_mesh(),
        scratch_types=[
            pltpu.VMEM((_DEG_DEPTH, _CHUNK), jnp.int32),
            pltpu.VMEM((_ROWS_A + 8,), jnp.float32),
            pltpu.VMEM_SHARED((N + _ACC_PAD_ROWS,), jnp.float32),
            pltpu.SemaphoreType.DMA((_DEG_DEPTH,)),
            pltpu.SemaphoreType.DMA,
        ],
    )
    def deg_kernel(eidx1_h, eidx2_h, deg1_h, deg2_h, dst_v, ones_v, acc,
                   isems, sem):
        c = lax.axis_index("c")
        s = lax.axis_index("s")

        def fill(i, _):
            ones_v[pl.ds(i * 16, 16)] = jnp.ones((16,), jnp.float32)
            return 0

        lax.fori_loop(0, (_ROWS_A + 8) // 16, fill, 0)

        def run(eidx_h, deg_h):
            def idx_start(g, j):
                pltpu.async_copy(_dst_slice(eidx_h, s, g), dst_v.at[j],
                                 isems.at[j])

            def idx_wait(g, j):
                pltpu.make_async_copy(_dst_slice(eidx_h, s, g), dst_v.at[j],
                                      isems.at[j]).wait()

            for j in range(_DEG_DEPTH):
                idx_start(j, j)
            _node_range_copy(
                s, lambda st, sz: pltpu.sync_copy(ones_v.at[pl.ds(0, sz)],
                                                  acc.at[pl.ds(st, sz)]))
            plsc.subcore_barrier()

            def body(k, _):
                for j in range(_DEG_DEPTH):
                    idx_wait(k * _DEG_DEPTH + j, j)
                descs = []
                for j in range(_DEG_DEPTH):
                    descs.append(pltpu.async_copy(
                        ones_v.at[pl.ds(0, _CHUNK)],
                        acc.at[dst_v.at[j]], sem, add=True))
                for d in descs:
                    d.wait()
                for j in range(_DEG_DEPTH):
                    idx_start((k + 1) * _DEG_DEPTH + j, j)
                return 0

            lax.fori_loop(0, _NMACRO, body, 0)
            for j in range(_DEG_DEPTH):
                idx_wait(_NMACRO * _DEG_DEPTH + j, j)
            plsc.subcore_barrier()

            def drain(st, sz):
                pltpu.sync_copy(acc.at[pl.ds(st, sz)], ones_v.at[pl.ds(0, sz)])
                pltpu.sync_copy(ones_v.at[pl.ds(0, sz)],
                                deg_h.at[pl.ds(st, sz)])

            _node_range_copy(s, drain)

        @pl.when(c == 0)
        def _():
            run(eidx1_h, deg1_h)

        @pl.when(c == 1)
        def _():
            run(eidx2_h, deg2_h)

    return deg_kernel(eidx1, eidx2)


def _sc_aggregate(eidx_a, eidx_b, vals_a, vals_b):
    @functools.partial(
        pl.kernel,
        out_type=[jax.ShapeDtypeStruct((N, D_OUT), jnp.float32)] * 2,
        mesh=_mesh(),
        scratch_types=[
            pltpu.VMEM((_IDX_SLOTS, _CHUNK), jnp.int32),
            pltpu.VMEM((_IDX_SLOTS, _CHUNK), jnp.int32),
            pltpu.VMEM((_DEPTH, _CHUNK, D_OUT), jnp.float32),
            pltpu.VMEM_SHARED((N + _ACC_PAD_ROWS, D_OUT), jnp.float32),
            pltpu.SemaphoreType.DMA((_IDX_SLOTS,)),
            pltpu.SemaphoreType.DMA((_DEPTH,)),
            pltpu.SemaphoreType.DMA((_DEPTH,)),
        ],
    )
    def agg_kernel(eidxa_h, eidxb_h, valsa_h, valsb_h,
                   outa_h, outb_h, src_v, dst_v, rows_v, acc, isems, gsems,
                   ssems):
        c = lax.axis_index("c")
        s = lax.axis_index("s")

        def run(eidx_h, vals_h, out_h):
            def idx_start(g, sl):
                pltpu.async_copy(_src_slice(eidx_h, s, g), src_v.at[sl],
                                 isems.at[sl])
                pltpu.async_copy(_dst_slice(eidx_h, s, g), dst_v.at[sl],
                                 isems.at[sl])

            def idx_wait(g, sl):
                pltpu.make_async_copy(_src_slice(eidx_h, s, g), src_v.at[sl],
                                      isems.at[sl]).wait()
                pltpu.make_async_copy(_dst_slice(eidx_h, s, g), dst_v.at[sl],
                                      isems.at[sl]).wait()

            def gather_start(sl6, sl3):
                pltpu.async_copy(vals_h.at[src_v.at[sl6]], rows_v.at[sl3],
                                 gsems.at[sl3])

            def gather_wait(sl6, sl3):
                pltpu.make_async_copy(vals_h.at[src_v.at[sl6]],
                                      rows_v.at[sl3], gsems.at[sl3]).wait()

            def scatter_start(sl6, sl3):
                pltpu.async_copy(rows_v.at[sl3], acc.at[dst_v.at[sl6]],
                                 ssems.at[sl3], add=True)

            def scatter_wait(sl6, sl3):
                pltpu.make_async_copy(rows_v.at[sl3], acc.at[dst_v.at[sl6]],
                                      ssems.at[sl3]).wait()

            for g in range(_IDX_SLOTS):
                idx_start(g, g)

            def chunked(st, sz, fn):
                off = 0
                while off < sz:
                    step = min(_CHUNK, sz - off)
                    fn(st + off, step)
                    off += step

            def init(st, sz):
                def one(o, n):
                    pltpu.sync_copy(vals_h.at[pl.ds(o, n)],
                                    rows_v.at[0, pl.ds(0, n)])
                    pltpu.sync_copy(rows_v.at[0, pl.ds(0, n)],
                                    acc.at[pl.ds(o, n)])
                chunked(st, sz, one)

            _node_range_copy(s, init)
            plsc.subcore_barrier()

            def position(g, j):
                scatter_wait((j + _DEPTH) % _IDX_SLOTS, j % _DEPTH)
                idx_start(g + _DEPTH, (j + _DEPTH) % _IDX_SLOTS)
                idx_wait(g, j)
                gather_start(j, j % _DEPTH)
                gather_wait((j - 1) % _IDX_SLOTS, (j - 1) % _DEPTH)
                scatter_start((j - 1) % _IDX_SLOTS, (j - 1) % _DEPTH)

            idx_wait(0, 0)
            gather_start(0, 0)
            for g in range(1, _DEPTH):
                idx_wait(g, g)
                gather_start(g, g)
                gather_wait(g - 1, g - 1)
                scatter_start(g - 1, g - 1)
            for g in range(_DEPTH, _IDX_SLOTS):
                position(g, g)

            def body(k, _):
                for j in range(_IDX_SLOTS):
                    position(k * _IDX_SLOTS + j, j)
                return 0

            lax.fori_loop(1, _NBODY, body, 0)

            last = _G_AGG - 1
            gather_wait(last % _IDX_SLOTS, last % _DEPTH)
            scatter_start(last % _IDX_SLOTS, last % _DEPTH)
            for g in range(_G_AGG - _DEPTH, _G_AGG):
                scatter_wait(g % _IDX_SLOTS, g % _DEPTH)
            for g in range(_G_AGG, _G_AGG + _DEPTH):
                idx_wait(g, g % _IDX_SLOTS)
            plsc.subcore_barrier()

            def drain(st, sz):
                def one(o, n):
                    pltpu.sync_copy(acc.at[pl.ds(o, n)],
                                    rows_v.at[0, pl.ds(0, n)])
                    pltpu.sync_copy(rows_v.at[0, pl.ds(0, n)],
                                    out_h.at[pl.ds(o, n)])
                chunked(st, sz, one)

            _node_range_copy(s, drain)

        @pl.when(c == 0)
        def _():
            run(eidxa_h, valsa_h, outa_h)

        @pl.when(c == 1)
        def _():
            run(eidxb_h, valsb_h, outb_h)

    return agg_kernel(eidx_a, eidx_b, vals_a, vals_b)


def _mm_body(x_ref, w_ref, h_ref):
    h_ref[...] = jnp.dot(x_ref[...], w_ref[...],
                         preferred_element_type=jnp.float32)


def _mm(x, W0):
    return pl.pallas_call(
        _mm_body,
        grid=(N // _ROW_BLOCK,),
        in_specs=[
            pl.BlockSpec((_ROW_BLOCK, D_IN), lambda i: (i, 0)),
            pl.BlockSpec((D_IN, D_HID), lambda i: (0, 0)),
        ],
        out_specs=pl.BlockSpec((_ROW_BLOCK, D_HID), lambda i: (i, 0)),
        out_shape=jax.ShapeDtypeStruct((N, D_HID), jnp.float32),
    )(x, W0)


def _scale_body(h_ref, deg_ref, hp0_ref, hp1_ref, dinv_ref):
    dinv = lax.rsqrt(deg_ref[...])
    hp = h_ref[...] * dinv
    hp0_ref[...] = hp[:, :D_OUT]
    hp1_ref[...] = hp[:, D_OUT:]
    dinv_ref[...] = dinv


def _scale(h, deg1):
    row_h = pl.BlockSpec((_ROW_BLOCK, D_OUT), lambda i: (i, 0))
    return pl.pallas_call(
        _scale_body,
        grid=(N // _ROW_BLOCK,),
        in_specs=[
            pl.BlockSpec((_ROW_BLOCK, D_HID), lambda i: (i, 0)),
            pl.BlockSpec((_ROW_BLOCK, 1), lambda i: (i, 0)),
        ],
        out_specs=[row_h, row_h, pl.BlockSpec((_ROW_BLOCK, 1), lambda i: (i, 0))],
        out_shape=[
            jax.ShapeDtypeStruct((N, D_OUT), jnp.float32),
            jax.ShapeDtypeStruct((N, D_OUT), jnp.float32),
            jax.ShapeDtypeStruct((N, 1), jnp.float32),
        ],
    )(h, deg1.reshape(N, 1))


def _mid_body(agg0_ref, agg1_ref, dinv1_ref, w1_ref, deg2_ref, b0_ref,
              gamma_ref, beta_ref, mean_ref, var_ref,
              emb_ref, hp21_ref, hp22_ref, dinv2_ref):
    agg = jnp.concatenate([agg0_ref[...], agg1_ref[...]], axis=1)
    conv = dinv1_ref[...] * agg + b0_ref[...]
    inv_std = lax.rsqrt(var_ref[...] + 1e-5)
    bn = (conv - mean_ref[...]) * inv_std * gamma_ref[...] + beta_ref[...]
    emb = jnp.maximum(bn, 0.0)
    emb_ref[...] = emb
    h2 = jnp.dot(emb, w1_ref[...], preferred_element_type=jnp.float32)
    dinv2 = lax.rsqrt(deg2_ref[...])
    hp21_ref[...] = h2 * dinv1_ref[...]
    hp22_ref[...] = h2 * dinv2
    dinv2_ref[...] = dinv2


def _mid(agg0, agg1, dinv1, W1, deg2, b0, gamma0, beta0, mean0, var0):
    row_h = pl.BlockSpec((_ROW_BLOCK, D_OUT), lambda i: (i, 0))
    col = pl.BlockSpec((_ROW_BLOCK, 1), lambda i: (i, 0))
    vec = pl.BlockSpec((1, D_HID), lambda i: (0, 0))
    v = lambda a: a.reshape(1, D_HID)
    return pl.pallas_call(
        _mid_body,
        grid=(N // _ROW_BLOCK,),
        in_specs=[row_h, row_h, col,
                  pl.BlockSpec((D_HID, D_OUT), lambda i: (0, 0)),
                  col, vec, vec, vec, vec, vec],
        out_specs=[pl.BlockSpec((_ROW_BLOCK, D_HID), lambda i: (i, 0)),
                   row_h, row_h, col],
        out_shape=[
            jax.ShapeDtypeStruct((N, D_HID), jnp.float32),
            jax.ShapeDtypeStruct((N, D_OUT), jnp.float32),
            jax.ShapeDtypeStruct((N, D_OUT), jnp.float32),
            jax.ShapeDtypeStruct((N, 1), jnp.float32),
        ],
    )(agg0, agg1, dinv1, W1, deg2.reshape(N, 1), v(b0), v(gamma0), v(beta0),
      v(mean0), v(var0))


def _final_body(agg2_ref, agg3_ref, dinv1_ref, dinv2_ref, b_ref,
                logits_ref, pred_ref, logits2_ref):
    raw1 = dinv1_ref[...] * agg2_ref[...] + b_ref[...]
    raw2 = dinv2_ref[...] * agg3_ref[...] + b_ref[...]
    m1 = jnp.max(raw1, axis=1, keepdims=True)
    e1 = jnp.exp(raw1 - m1)
    logits_ref[...] = e1 / jnp.sum(e1, axis=1, keepdims=True)
    m2 = jnp.max(raw2, axis=1, keepdims=True)
    e2 = jnp.exp(raw2 - m2)
    logits2_ref[...] = e2 / jnp.sum(e2, axis=1, keepdims=True)
    pred_ref[...] = jnp.argmax(raw1, axis=1, keepdims=True).astype(jnp.int32)


def _final(agg2, agg3, dinv1, dinv2, b1):
    row_h = pl.BlockSpec((_ROW_BLOCK, D_OUT), lambda i: (i, 0))
    col = pl.BlockSpec((_ROW_BLOCK, 1), lambda i: (i, 0))
    logits, pred, logits2 = pl.pallas_call(
        _final_body,
        grid=(N // _ROW_BLOCK,),
        in_specs=[row_h, row_h, col, col,
                  pl.BlockSpec((1, D_OUT), lambda i: (0, 0))],
        out_specs=[row_h, col, row_h],
        out_shape=[
            jax.ShapeDtypeStruct((N, D_OUT), jnp.float32),
            jax.ShapeDtypeStruct((N, 1), jnp.int32),
            jax.ShapeDtypeStruct((N, D_OUT), jnp.float32),
        ],
    )(agg2, agg3, dinv1, dinv2, b1.reshape(1, D_OUT))
    return logits, pred.reshape(N), logits2


def kernel(x, adj_t, adj_t2, W0, b0, gamma0, beta0, bn_mean0, bn_var0, W1, b1):
    eidx1 = _pad_edges(adj_t)
    eidx2 = _pad_edges(adj_t2)
    h = _mm(x, W0)
    deg1, deg2 = _sc_degrees(eidx1, eidx2)
    hp0, hp1, dinv1 = _scale(h, deg1)
    agg0, agg1 = _sc_aggregate(eidx1, eidx1, hp0, hp1)
    emb, hp2_1, hp2_2, dinv2 = _mid(agg0, agg1, dinv1, W1, deg2, b0, gamma0,
                                    beta0, bn_mean0, bn_var0)
    agg2, agg3 = _sc_aggregate(eidx1, eidx2, hp2_1, hp2_2)
    logits, predictions, logits_2 = _final(agg2, agg3, dinv1, dinv2, b1)
    return (logits, predictions, logits_2, emb)


# speedup vs baseline: 1.1231x; 1.1231x over previous
import functools

import jax
import jax.numpy as jnp
from jax import lax
from jax.experimental import pallas as pl
from jax.experimental.pallas import tpu as pltpu
from jax.experimental.pallas import tpu_sc as plsc

N = 10000
D_IN = 256
D_HID = 256
D_OUT = 128
E = 160000

_CHUNK = 120
_NSUB = 16
_DEPTH = 3
_IDX_SLOTS = 2 * _DEPTH
_G_AGG = 84
_NBODY = _G_AGG // _IDX_SLOTS
_DEG_DEPTH = 3
_NMACRO = 28
_G_IDX = _G_AGG + _DEPTH
_PAD = _NSUB * _G_IDX * _CHUNK - E
_ACC_PAD_ROWS = 8
_ROWS_A = 632
_ROWS_LAST = N - 15 * _ROWS_A

_ROW_BLOCK = 1000


def _pad_edges(adj):
    flat = adj.reshape(2 * E)
    i = jnp.arange(_PAD, dtype=jnp.int32)
    return flat, jnp.concatenate([flat[E:], N + (i % _ACC_PAD_ROWS)])


def _idx_slice(h, s, g):
    return h.at[pl.ds((g * _NSUB + s) * _CHUNK, _CHUNK)]


def _mesh():
    return plsc.VectorSubcoreMesh(core_axis_name="c", subcore_axis_name="s")


def _node_range_copy(s, copy_fn):
    @pl.when(s < 15)
    def _():
        copy_fn(s * _ROWS_A, _ROWS_A)

    @pl.when(s == 15)
    def _():
        copy_fn(15 * _ROWS_A, _ROWS_LAST)


def _sc_degrees(dst1, dst2):
    @functools.partial(
        pl.kernel,
        out_type=[jax.ShapeDtypeStruct((N,), jnp.float32)] * 2,
        mesh=_mesh(),
        scratch_types=[
            pltpu.VMEM((_DEG_DEPTH, _CHUNK), jnp.int32),
            pltpu.VMEM((_ROWS_A + 8,), jnp.float32),
            pltpu.VMEM_SHARED((N + _ACC_PAD_ROWS,), jnp.float32),
            pltpu.SemaphoreType.DMA((_DEG_DEPTH,)),
            pltpu.SemaphoreType.DMA,
        ],
    )
    def deg_kernel(dst1_h, dst2_h, deg1_h, deg2_h, dst_v, ones_v, acc,
                   isems, sem):
        c = lax.axis_index("c")
        s = lax.axis_index("s")

        def fill(i, _):
            ones_v[pl.ds(i * 16, 16)] = jnp.ones((16,), jnp.float32)
            return 0

        lax.fori_loop(0, (_ROWS_A + 8) // 16, fill, 0)

        def run(dst_h, deg_h):
            def idx_start(g, j):
                pltpu.async_copy(_idx_slice(dst_h, s, g), dst_v.at[j],
                                 isems.at[j])

            def idx_wait(g, j):
                pltpu.make_async_copy(_idx_slice(dst_h, s, g), dst_v.at[j],
                                      isems.at[j]).wait()

            for j in range(_DEG_DEPTH):
                idx_start(j, j)
            _node_range_copy(
                s, lambda st, sz: pltpu.sync_copy(ones_v.at[pl.ds(0, sz)],
                                                  acc.at[pl.ds(st, sz)]))
            plsc.subcore_barrier()

            def body(k, _):
                for j in range(_DEG_DEPTH):
                    idx_wait(k * _DEG_DEPTH + j, j)
                descs = []
                for j in range(_DEG_DEPTH):
                    descs.append(pltpu.async_copy(
                        ones_v.at[pl.ds(0, _CHUNK)],
                        acc.at[dst_v.at[j]], sem, add=True))
                for d in descs:
                    d.wait()
                for j in range(_DEG_DEPTH):
                    idx_start((k + 1) * _DEG_DEPTH + j, j)
                return 0

            lax.fori_loop(0, _NMACRO, body, 0)
            for j in range(_DEG_DEPTH):
                idx_wait(_NMACRO * _DEG_DEPTH + j, j)
            plsc.subcore_barrier()

            def drain(st, sz):
                pltpu.sync_copy(acc.at[pl.ds(st, sz)], ones_v.at[pl.ds(0, sz)])
                pltpu.sync_copy(ones_v.at[pl.ds(0, sz)],
                                deg_h.at[pl.ds(st, sz)])

            _node_range_copy(s, drain)

        @pl.when(c == 0)
        def _():
            run(dst1_h, deg1_h)

        @pl.when(c == 1)
        def _():
            run(dst2_h, deg2_h)

    return deg_kernel(dst1, dst2)


def _sc_aggregate(src_a, dst_a, src_b, dst_b, vals_a, vals_b):
    @functools.partial(
        pl.kernel,
        out_type=[jax.ShapeDtypeStruct((N, D_OUT), jnp.float32)] * 2,
        mesh=_mesh(),
        scratch_types=[
            pltpu.VMEM((_IDX_SLOTS, _CHUNK), jnp.int32),
            pltpu.VMEM((_IDX_SLOTS, _CHUNK), jnp.int32),
            pltpu.VMEM((_DEPTH, _CHUNK, D_OUT), jnp.float32),
            pltpu.VMEM_SHARED((N + _ACC_PAD_ROWS, D_OUT), jnp.float32),
            pltpu.SemaphoreType.DMA((_IDX_SLOTS,)),
            pltpu.SemaphoreType.DMA((_DEPTH,)),
            pltpu.SemaphoreType.DMA((_DEPTH,)),
        ],
    )
    def agg_kernel(srca_h, dsta_h, srcb_h, dstb_h, valsa_h, valsb_h,
                   outa_h, outb_h, src_v, dst_v, rows_v, acc, isems, gsems,
                   ssems):
        c = lax.axis_index("c")
        s = lax.axis_index("s")

        def run(src_h, dst_h, vals_h, out_h):
            def idx_start(g, sl):
                pltpu.async_copy(_idx_slice(src_h, s, g), src_v.at[sl],
                                 isems.at[sl])
                pltpu.async_copy(_idx_slice(dst_h, s, g), dst_v.at[sl],
                                 isems.at[sl])

            def idx_wait(g, sl):
                pltpu.make_async_copy(_idx_slice(src_h, s, g), src_v.at[sl],
                                      isems.at[sl]).wait()
                pltpu.make_async_copy(_idx_slice(dst_h, s, g), dst_v.at[sl],
                                      isems.at[sl]).wait()

            def gather_start(sl6, sl3):
                pltpu.async_copy(vals_h.at[src_v.at[sl6]], rows_v.at[sl3],
                                 gsems.at[sl3])

            def gather_wait(sl6, sl3):
                pltpu.make_async_copy(vals_h.at[src_v.at[sl6]],
                                      rows_v.at[sl3], gsems.at[sl3]).wait()

            def scatter_start(sl6, sl3):
                pltpu.async_copy(rows_v.at[sl3], acc.at[dst_v.at[sl6]],
                                 ssems.at[sl3], add=True)

            def scatter_wait(sl6, sl3):
                pltpu.make_async_copy(rows_v.at[sl3], acc.at[dst_v.at[sl6]],
                                      ssems.at[sl3]).wait()

            for g in range(_IDX_SLOTS):
                idx_start(g, g)

            def chunked(st, sz, fn):
                off = 0
                while off < sz:
                    step = min(_CHUNK, sz - off)
                    fn(st + off, step)
                    off += step

            def init(st, sz):
                def one(o, n):
                    pltpu.sync_copy(vals_h.at[pl.ds(o, n)],
                                    rows_v.at[0, pl.ds(0, n)])
                    pltpu.sync_copy(rows_v.at[0, pl.ds(0, n)],
                                    acc.at[pl.ds(o, n)])
                chunked(st, sz, one)

            _node_range_copy(s, init)
            plsc.subcore_barrier()

            def position(g, j):
                scatter_wait((j + _DEPTH) % _IDX_SLOTS, j % _DEPTH)
                idx_start(g + _DEPTH, (j + _DEPTH) % _IDX_SLOTS)
                idx_wait(g, j)
                gather_start(j, j % _DEPTH)
                gather_wait((j - 1) % _IDX_SLOTS, (j - 1) % _DEPTH)
                scatter_start((j - 1) % _IDX_SLOTS, (j - 1) % _DEPTH)

            idx_wait(0, 0)
            gather_start(0, 0)
            for g in range(1, _DEPTH):
                idx_wait(g, g)
                gather_start(g, g)
                gather_wait(g - 1, g - 1)
                scatter_start(g - 1, g - 1)
            for g in range(_DEPTH, _IDX_SLOTS):
                position(g, g)

            def body(k, _):
                for j in range(_IDX_SLOTS):
                    position(k * _IDX_SLOTS + j, j)
                return 0

            lax.fori_loop(1, _NBODY, body, 0)

            last = _G_AGG - 1
            gather_wait(last % _IDX_SLOTS, last % _DEPTH)
            scatter_start(last % _IDX_SLOTS, last % _DEPTH)
            for g in range(_G_AGG - _DEPTH, _G_AGG):
                scatter_wait(g % _IDX_SLOTS, g % _DEPTH)
            for g in range(_G_AGG, _G_AGG + _DEPTH):
                idx_wait(g, g % _IDX_SLOTS)
            plsc.subcore_barrier()

            def drain(st, sz):
                def one(o, n):
                    pltpu.sync_copy(acc.at[pl.ds(o, n)],
                                    rows_v.at[0, pl.ds(0, n)])
                    pltpu.sync_copy(rows_v.at[0, pl.ds(0, n)],
                                    out_h.at[pl.ds(o, n)])
                chunked(st, sz, one)

            _node_range_copy(s, drain)

        @pl.when(c == 0)
        def _():
            run(srca_h, dsta_h, valsa_h, outa_h)

        @pl.when(c == 1)
        def _():
            run(srcb_h, dstb_h, valsb_h, outb_h)

    return agg_kernel(src_a, dst_a, src_b, dst_b, vals_a, vals_b)


def _mm_body(x_ref, w_ref, h_ref):
    h_ref[...] = jnp.dot(x_ref[...], w_ref[...],
                         preferred_element_type=jnp.float32)


def _mm(x, W0):
    return pl.pallas_call(
        _mm_body,
        grid=(N // _ROW_BLOCK,),
        in_specs=[
            pl.BlockSpec((_ROW_BLOCK, D_IN), lambda i: (i, 0)),
            pl.BlockSpec((D_IN, D_HID), lambda i: (0, 0)),
        ],
        out_specs=pl.BlockSpec((_ROW_BLOCK, D_HID), lambda i: (i, 0)),
        out_shape=jax.ShapeDtypeStruct((N, D_HID), jnp.float32),
    )(x, W0)


def _scale_body(h_ref, deg_ref, hp0_ref, hp1_ref, dinv_ref):
    dinv = lax.rsqrt(deg_ref[...])
    hp = h_ref[...] * dinv
    hp0_ref[...] = hp[:, :D_OUT]
    hp1_ref[...] = hp[:, D_OUT:]
    dinv_ref[...] = dinv


def _scale(h, deg1):
    row_h = pl.BlockSpec((_ROW_BLOCK, D_OUT), lambda i: (i, 0))
    return pl.pallas_call(
        _scale_body,
        grid=(N // _ROW_BLOCK,),
        in_specs=[
            pl.BlockSpec((_ROW_BLOCK, D_HID), lambda i: (i, 0)),
            pl.BlockSpec((_ROW_BLOCK, 1), lambda i: (i, 0)),
        ],
        out_specs=[row_h, row_h, pl.BlockSpec((_ROW_BLOCK, 1), lambda i: (i, 0))],
        out_shape=[
            jax.ShapeDtypeStruct((N, D_OUT), jnp.float32),
            jax.ShapeDtypeStruct((N, D_OUT), jnp.float32),
            jax.ShapeDtypeStruct((N, 1), jnp.float32),
        ],
    )(h, deg1.reshape(N, 1))


def _mid_body(agg0_ref, agg1_ref, dinv1_ref, w1_ref, deg2_ref, b0_ref,
              gamma_ref, beta_ref, mean_ref, var_ref,
              emb_ref, hp21_ref, hp22_ref, dinv2_ref):
    agg = jnp.concatenate([agg0_ref[...], agg1_ref[...]], axis=1)
    conv = dinv1_ref[...] * agg + b0_ref[...]
    inv_std = lax.rsqrt(var_ref[...] + 1e-5)
    bn = (conv - mean_ref[...]) * inv_std * gamma_ref[...] + beta_ref[...]
    emb = jnp.maximum(bn, 0.0)
    emb_ref[...] = emb
    h2 = jnp.dot(emb, w1_ref[...], preferred_element_type=jnp.float32)
    dinv2 = lax.rsqrt(deg2_ref[...])
    hp21_ref[...] = h2 * dinv1_ref[...]
    hp22_ref[...] = h2 * dinv2
    dinv2_ref[...] = dinv2


def _mid(agg0, agg1, dinv1, W1, deg2, b0, gamma0, beta0, mean0, var0):
    row_h = pl.BlockSpec((_ROW_BLOCK, D_OUT), lambda i: (i, 0))
    col = pl.BlockSpec((_ROW_BLOCK, 1), lambda i: (i, 0))
    vec = pl.BlockSpec((1, D_HID), lambda i: (0, 0))
    v = lambda a: a.reshape(1, D_HID)
    return pl.pallas_call(
        _mid_body,
        grid=(N // _ROW_BLOCK,),
        in_specs=[row_h, row_h, col,
                  pl.BlockSpec((D_HID, D_OUT), lambda i: (0, 0)),
                  col, vec, vec, vec, vec, vec],
        out_specs=[pl.BlockSpec((_ROW_BLOCK, D_HID), lambda i: (i, 0)),
                   row_h, row_h, col],
        out_shape=[
            jax.ShapeDtypeStruct((N, D_HID), jnp.float32),
            jax.ShapeDtypeStruct((N, D_OUT), jnp.float32),
            jax.ShapeDtypeStruct((N, D_OUT), jnp.float32),
            jax.ShapeDtypeStruct((N, 1), jnp.float32),
        ],
    )(agg0, agg1, dinv1, W1, deg2.reshape(N, 1), v(b0), v(gamma0), v(beta0),
      v(mean0), v(var0))


def _final_body(agg2_ref, agg3_ref, dinv1_ref, dinv2_ref, b_ref,
                logits_ref, pred_ref, logits2_ref):
    raw1 = dinv1_ref[...] * agg2_ref[...] + b_ref[...]
    raw2 = dinv2_ref[...] * agg3_ref[...] + b_ref[...]
    m1 = jnp.max(raw1, axis=1, keepdims=True)
    e1 = jnp.exp(raw1 - m1)
    logits_ref[...] = e1 / jnp.sum(e1, axis=1, keepdims=True)
    m2 = jnp.max(raw2, axis=1, keepdims=True)
    e2 = jnp.exp(raw2 - m2)
    logits2_ref[...] = e2 / jnp.sum(e2, axis=1, keepdims=True)
    pred_ref[...] = jnp.argmax(raw1, axis=1, keepdims=True).astype(jnp.int32)


def _final(agg2, agg3, dinv1, dinv2, b1):
    row_h = pl.BlockSpec((_ROW_BLOCK, D_OUT), lambda i: (i, 0))
    col = pl.BlockSpec((_ROW_BLOCK, 1), lambda i: (i, 0))
    logits, pred, logits2 = pl.pallas_call(
        _final_body,
        grid=(N // _ROW_BLOCK,),
        in_specs=[row_h, row_h, col, col,
                  pl.BlockSpec((1, D_OUT), lambda i: (0, 0))],
        out_specs=[row_h, col, row_h],
        out_shape=[
            jax.ShapeDtypeStruct((N, D_OUT), jnp.float32),
            jax.ShapeDtypeStruct((N, 1), jnp.int32),
            jax.ShapeDtypeStruct((N, D_OUT), jnp.float32),
        ],
    )(agg2, agg3, dinv1, dinv2, b1.reshape(1, D_OUT))
    return logits, pred.reshape(N), logits2


def kernel(x, adj_t, adj_t2, W0, b0, gamma0, beta0, bn_mean0, bn_var0, W1, b1):
    src1, dst1 = _pad_edges(adj_t)
    src2, dst2 = _pad_edges(adj_t2)
    h = _mm(x, W0)
    deg1, deg2 = _sc_degrees(dst1, dst2)
    hp0, hp1, dinv1 = _scale(h, deg1)
    agg0, agg1 = _sc_aggregate(src1, dst1, src1, dst1, hp0, hp1)
    emb, hp2_1, hp2_2, dinv2 = _mid(agg0, agg1, dinv1, W1, deg2, b0, gamma0,
                                    beta0, bn_mean0, bn_var0)
    agg2, agg3 = _sc_aggregate(src1, dst1, src2, dst2, hp2_1, hp2_2)
    logits, predictions, logits_2 = _final(agg2, agg3, dinv1, dinv2, b1)
    return (logits, predictions, logits_2, emb)
